# Initial kernel scaffold; baseline (speedup 1.0000x reference)
import jax
import jax.numpy as jnp
from jax.experimental import pallas as pl


def kernel(x):
    raise NotImplementedError("write your pallas kernel here")


# speedup vs baseline: 15.8138x; 15.8138x over previous
import jax
import jax.numpy as jnp
from jax.experimental import pallas as pl

_HW = 112 * 112
_K = int(0.5 * _HW)
_MAXBITS = 0x7F800000
_BP = 16


def _body(x_ref, o_ref):
    x = x_ref[...]
    bits = jax.lax.bitcast_convert_type(x, jnp.int32) & 0x7FFFFFFF

    lo = jnp.zeros((_BP, 1), jnp.int32)
    hi = jnp.full((_BP, 1), _MAXBITS, jnp.int32)

    def it(_, carry):
        lo, hi = carry
        mid = lo + ((hi - lo + 1) >> 1)
        cnt = jnp.sum((bits >= mid).astype(jnp.int32), axis=1, keepdims=True)
        ok = cnt >= _K
        return jnp.where(ok, mid, lo), jnp.where(ok, hi, mid - 1)

    lo, _ = jax.lax.fori_loop(0, 31, it, (lo, hi))
    o_ref[...] = jnp.where(bits >= lo, x, 0.0)


def kernel(x):
    B, C, H, W = x.shape
    planes = B * C
    x2 = x.reshape(planes, H * W)
    out = pl.pallas_call(
        _body,
        grid=(planes // _BP,),
        in_specs=[pl.BlockSpec((_BP, H * W), lambda i: (i, 0))],
        out_specs=pl.BlockSpec((_BP, H * W), lambda i: (i, 0)),
        out_shape=jax.ShapeDtypeStruct((planes, H * W), x.dtype),
    )(x2)
    return out.reshape(B, C, H, W)
